# baseline (speedup 1.0000x reference)
import functools

import jax
import jax.numpy as jnp
from jax import lax
from jax.experimental import pallas as pl
from jax.experimental.pallas import tpu as pltpu
from jax.experimental.pallas import tpu_sc as plsc

FB = 128
BM = 5000
CH = 125
N_TILES = 16
N_CORES = 2
BN_EPS = 1e-5


def _sc_agg(hT, srcp, dstr, zeros, *, n_nodes, n_chunks, m):
    nfb = srcp.shape[0]
    rows_per_tile = (-(-n_nodes // N_TILES) + 7) // 8 * 8
    last_r0 = n_nodes - rows_per_tile
    mesh = plsc.VectorSubcoreMesh(core_axis_name="c", subcore_axis_name="s")

    @functools.partial(
        pl.kernel,
        mesh=mesh,
        out_type=jax.ShapeDtypeStruct((N_CORES, n_nodes, FB), jnp.float32),
        scratch_types=[
            pltpu.VMEM((n_chunks, CH), jnp.int32),
            pltpu.VMEM((n_chunks, CH), jnp.int32),
            pltpu.VMEM((CH, FB), jnp.float32),
            pltpu.VMEM_SHARED((n_nodes, FB), jnp.float32),
            pltpu.SemaphoreType.DMA,
        ],
    )
    def agg(hT_hbm, srcp_hbm, dst_hbm, z_hbm, out_hbm,
            src_v, dst_v, rows, acc, gsem):
        c = lax.axis_index("c")
        s = lax.axis_index("s")
        r0 = pl.multiple_of(jnp.minimum(s * rows_per_tile, last_r0), 8)
        pltpu.sync_copy(dst_hbm.at[s], dst_v)
        for bi in range(1):
            b = 2 * c + m
            pltpu.sync_copy(z_hbm.at[pl.ds(r0, rows_per_tile)],
                            acc.at[pl.ds(r0, rows_per_tile)])
            pltpu.sync_copy(srcp_hbm.at[b, s], src_v)
            plsc.subcore_barrier()

            def body(j, carry):
                pltpu.async_copy(hT_hbm.at[src_v.at[j]], rows, gsem).wait()
                pltpu.sync_copy(rows, acc.at[dst_v.at[j]], add=True)
                return carry

            lax.fori_loop(0, n_chunks, body, 0)
            plsc.subcore_barrier()
            pltpu.sync_copy(acc.at[pl.ds(r0, rows_per_tile)],
                            out_hbm.at[c, pl.ds(r0, rows_per_tile)])
            plsc.subcore_barrier()

    return agg(hT, srcp, dstr, zeros)


def _embed(x, w, b, *, n_nodes):
    nmb = n_nodes // BM
    nfb = w.shape[1] // FB

    def body(x_ref, w_ref, b_ref, o_ref, cs_ref):
        j, i = pl.program_id(0), pl.program_id(1)
        y = jnp.dot(x_ref[...].astype(jnp.bfloat16),
                    w_ref[...].astype(jnp.bfloat16),
                    preferred_element_type=jnp.float32) + b_ref[...]
        y0, y1 = y[:, :FB], y[:, FB:]
        o_ref[...] = jnp.stack([y0, y1])

        @pl.when(i == 0)
        def _():
            cs_ref[...] = jnp.zeros_like(cs_ref)

        cs_ref[...] += jnp.sum(y, axis=0, keepdims=True)

    return pl.pallas_call(
        body,
        grid=(N_CORES, nmb),
        in_specs=[
            pl.BlockSpec((BM, x.shape[1]), lambda j, i: (i, 0)),
            pl.BlockSpec((x.shape[1], 2 * FB), lambda j, i: (0, j)),
            pl.BlockSpec((1, 2 * FB), lambda j, i: (0, j)),
        ],
        out_specs=[
            pl.BlockSpec((2, BM, FB), lambda j, i: (j, i, 0)),
            pl.BlockSpec((1, 2 * FB), lambda j, i: (0, j)),
        ],
        out_shape=[
            jax.ShapeDtypeStruct((nfb, n_nodes, FB), jnp.float32),
            jax.ShapeDtypeStruct((1, nfb * FB), jnp.float32),
        ],
    )(x, w, b.reshape(1, -1))


def _k1_partial(h_blk, neigh_m, eps, wr, *, n_nodes, m, hid):
    BM = 2000
    nmb = n_nodes // BM

    def body(h_ref, n_ref, e_ref, w_ref, y_ref):
        i, j2 = pl.program_id(0), pl.program_id(1)
        a = (1.0 + e_ref[0, 0]) * h_ref[0] + n_ref[0]
        part = jnp.dot(a.astype(jnp.bfloat16), w_ref[0].astype(jnp.bfloat16),
                       preferred_element_type=jnp.float32)

        @pl.when(j2 == 0)
        def _():
            y_ref[...] = part

        @pl.when(j2 > 0)
        def _():
            y_ref[...] += part

    return pl.pallas_call(
        body,
        grid=(nmb, N_CORES),
        in_specs=[
            pl.BlockSpec((1, BM, FB), lambda i, j2: (m + 2 * j2, i, 0)),
            pl.BlockSpec((1, BM, FB), lambda i, j2: (j2, i, 0)),
            pl.BlockSpec((1, 1), lambda i, j2: (0, 0),
                         memory_space=pltpu.SMEM),
            pl.BlockSpec((1, FB, hid), lambda i, j2: (m + 2 * j2, 0, 0)),
        ],
        out_specs=pl.BlockSpec((BM, hid), lambda i, j2: (i, 0)),
        out_shape=jax.ShapeDtypeStruct((n_nodes, hid), jnp.float32),
    )(h_blk, neigh_m, eps.reshape(1, 1), wr)


def _k1_final(y1a, h_blk, neigh_m, eps, wr, b, *, n_nodes, m, hid):
    BM = 2000
    nmb = n_nodes // BM

    def body(ya_ref, h_ref, n_ref, e_ref, w_ref, b_ref, y_ref, st_ref):
        i, j2 = pl.program_id(0), pl.program_id(1)
        a = (1.0 + e_ref[0, 0]) * h_ref[0] + n_ref[0]
        part = jnp.dot(a.astype(jnp.bfloat16), w_ref[0].astype(jnp.bfloat16),
                       preferred_element_type=jnp.float32)

        @pl.when(j2 == 0)
        def _():
            y_ref[...] = ya_ref[...] + part + b_ref[...]

        @pl.when(j2 > 0)
        def _():
            y_ref[...] += part

        @pl.when(jnp.logical_and(i == 0, j2 == 0))
        def _():
            st_ref[...] = jnp.zeros_like(st_ref)

        @pl.when(j2 == N_CORES - 1)
        def _():
            y = y_ref[...]
            st_ref[0:1, :] += jnp.sum(y, axis=0, keepdims=True)
            st_ref[1:2, :] += jnp.sum(y * y, axis=0, keepdims=True)

    return pl.pallas_call(
        body,
        grid=(nmb, N_CORES),
        in_specs=[
            pl.BlockSpec((BM, hid), lambda i, j2: (i, 0)),
            pl.BlockSpec((1, BM, FB), lambda i, j2: (m + 2 * j2, i, 0)),
            pl.BlockSpec((1, BM, FB), lambda i, j2: (j2, i, 0)),
            pl.BlockSpec((1, 1), lambda i, j2: (0, 0),
                         memory_space=pltpu.SMEM),
            pl.BlockSpec((1, FB, hid), lambda i, j2: (m + 2 * j2, 0, 0)),
            pl.BlockSpec((1, hid), lambda i, j2: (0, 0)),
        ],
        out_specs=[
            pl.BlockSpec((BM, hid), lambda i, j2: (i, 0)),
            pl.BlockSpec((2, hid), lambda i, j2: (0, 0)),
        ],
        out_shape=[
            jax.ShapeDtypeStruct((n_nodes, hid), jnp.float32),
            jax.ShapeDtypeStruct((2, hid), jnp.float32),
        ],
        input_output_aliases={0: 0},
    )(y1a, h_blk, neigh_m, eps.reshape(1, 1), wr, b.reshape(1, -1))


def _bn_relu_matmul(y1, st1, g1, b1, w, b, *, n_nodes):
    hid = w.shape[1]
    nmb = n_nodes // BM

    def body(y1_ref, st_ref, g_ref, bb_ref, w_ref, b_ref, y_ref, st2_ref):
        i = pl.program_id(0)
        n = jnp.float32(n_nodes)
        mean = st_ref[0:1, :] / n
        var = st_ref[1:2, :] / n - mean * mean
        scale = g_ref[...] * lax.rsqrt(var + BN_EPS)
        shift = bb_ref[...] - mean * scale
        z = jnp.maximum(y1_ref[...] * scale + shift, 0.0)
        y = jnp.dot(z.astype(jnp.bfloat16), w_ref[...].astype(jnp.bfloat16),
                    preferred_element_type=jnp.float32) + b_ref[...]
        y_ref[...] = y

        @pl.when(i == 0)
        def _():
            st2_ref[...] = jnp.zeros_like(st2_ref)

        st2_ref[0:1, :] += jnp.sum(y, axis=0, keepdims=True)
        st2_ref[1:2, :] += jnp.sum(y * y, axis=0, keepdims=True)

    return pl.pallas_call(
        body,
        grid=(nmb,),
        in_specs=[
            pl.BlockSpec((BM, hid), lambda i: (i, 0)),
            pl.BlockSpec((2, hid), lambda i: (0, 0)),
            pl.BlockSpec((1, hid), lambda i: (0, 0)),
            pl.BlockSpec((1, hid), lambda i: (0, 0)),
            pl.BlockSpec((hid, hid), lambda i: (0, 0)),
            pl.BlockSpec((1, hid), lambda i: (0, 0)),
        ],
        out_specs=[
            pl.BlockSpec((BM, hid), lambda i: (i, 0)),
            pl.BlockSpec((2, hid), lambda i: (0, 0)),
        ],
        out_shape=[
            jax.ShapeDtypeStruct((n_nodes, hid), jnp.float32),
            jax.ShapeDtypeStruct((2, hid), jnp.float32),
        ],
    )(y1, st1, g1.reshape(1, -1), b1.reshape(1, -1), w, b.reshape(1, -1))


def _bn_relu_snorm(y2, st2, g2, b2, snorm, *, n_nodes):
    hid = y2.shape[1]
    nfb = hid // FB
    nmb = n_nodes // BM

    def body(y_ref, st_ref, g_ref, bb_ref, sn_ref, t_ref, st3_ref):
        j, i = pl.program_id(0), pl.program_id(1)
        n = jnp.float32(n_nodes)
        mean = st_ref[0:1, :] / n
        var = st_ref[1:2, :] / n - mean * mean
        scale = g_ref[...] * lax.rsqrt(var + BN_EPS)
        shift = bb_ref[...] - mean * scale
        t = jnp.maximum(y_ref[...] * scale + shift, 0.0) * sn_ref[...]
        t_ref[...] = t[None]

        @pl.when(i == 0)
        def _():
            st3_ref[...] = jnp.zeros_like(st3_ref)

        st3_ref[0:1, :] += jnp.sum(t, axis=0, keepdims=True)
        st3_ref[1:2, :] += jnp.sum(t * t, axis=0, keepdims=True)

    return pl.pallas_call(
        body,
        grid=(nfb, nmb),
        in_specs=[
            pl.BlockSpec((BM, FB), lambda j, i: (i, j)),
            pl.BlockSpec((2, FB), lambda j, i: (0, j)),
            pl.BlockSpec((1, FB), lambda j, i: (0, j)),
            pl.BlockSpec((1, FB), lambda j, i: (0, j)),
            pl.BlockSpec((BM, 1), lambda j, i: (i, 0)),
        ],
        out_specs=[
            pl.BlockSpec((1, BM, FB), lambda j, i: (j, i, 0)),
            pl.BlockSpec((2, FB), lambda j, i: (0, j)),
        ],
        out_shape=[
            jax.ShapeDtypeStruct((nfb, n_nodes, FB), jnp.float32),
            jax.ShapeDtypeStruct((2, hid), jnp.float32),
        ],
    )(y2, st2, g2.reshape(1, -1), b2.reshape(1, -1), snorm)


def _bn_relu_residual(t_blk, st3, g3, b3, h_blk, *, n_nodes):
    nfb = t_blk.shape[0]
    nmb = n_nodes // BM

    def body(t_ref, st_ref, g_ref, bb_ref, h_ref, o_ref, cs_ref):
        j, i = pl.program_id(0), pl.program_id(1)
        n = jnp.float32(n_nodes)
        mean = st_ref[0:1, :] / n
        var = st_ref[1:2, :] / n - mean * mean
        scale = g_ref[...] * lax.rsqrt(var + BN_EPS)
        shift = bb_ref[...] - mean * scale
        h0 = h_ref[0] + jnp.maximum(
            t_ref[0] * scale[:, :FB] + shift[:, :FB], 0.0)
        h1 = h_ref[1] + jnp.maximum(
            t_ref[1] * scale[:, FB:] + shift[:, FB:], 0.0)
        o_ref[...] = jnp.stack([h0, h1])

        @pl.when(i == 0)
        def _():
            cs_ref[...] = jnp.zeros_like(cs_ref)

        cs_ref[...] += jnp.concatenate(
            [jnp.sum(h0, axis=0, keepdims=True),
             jnp.sum(h1, axis=0, keepdims=True)], axis=1)

    return pl.pallas_call(
        body,
        grid=(N_CORES, nmb),
        in_specs=[
            pl.BlockSpec((2, BM, FB), lambda j, i: (j, i, 0)),
            pl.BlockSpec((2, 2 * FB), lambda j, i: (0, j)),
            pl.BlockSpec((1, 2 * FB), lambda j, i: (0, j)),
            pl.BlockSpec((1, 2 * FB), lambda j, i: (0, j)),
            pl.BlockSpec((2, BM, FB), lambda j, i: (j, i, 0)),
        ],
        out_specs=[
            pl.BlockSpec((2, BM, FB), lambda j, i: (j, i, 0)),
            pl.BlockSpec((1, 2 * FB), lambda j, i: (0, j)),
        ],
        out_shape=[
            jax.ShapeDtypeStruct((nfb, n_nodes, FB), jnp.float32),
            jax.ShapeDtypeStruct((1, nfb * FB), jnp.float32),
        ],
    )(t_blk, st3, g3.reshape(1, -1), b3.reshape(1, -1), h_blk)


def _score(pooled_cat, w_cat, b_all):
    ncls = w_cat.shape[1]

    def body(p_ref, w_ref, b_ref, o_ref):
        o_ref[...] = (
            jnp.dot(p_ref[...], w_ref[...], preferred_element_type=jnp.float32)
            + jnp.sum(b_ref[...], axis=0, keepdims=True))

    return pl.pallas_call(
        body,
        out_shape=jax.ShapeDtypeStruct((1, ncls), jnp.float32),
    )(pooled_cat, w_cat, b_all)


def kernel(h, edge_index, e, snorm_n, snorm_e, params):
    n_nodes = h.shape[0]
    n_edges = edge_index.shape[1]
    hid = params["embed"]["W"].shape[1]
    nfb = hid // FB
    ept = n_edges // N_TILES
    n_chunks = ept // CH

    src = edge_index[0]
    dst = edge_index[1]
    offs = (jnp.arange(nfb, dtype=jnp.int32) * n_nodes)[:, None]
    srcp = (src[None, :] + offs).reshape(nfb, N_TILES, n_chunks, CH)
    dstr = dst.reshape(N_TILES, n_chunks, CH)
    zeros = jnp.zeros((n_nodes, FB), jnp.float32)

    h_blk, cs0 = _embed(h, params["embed"]["W"], params["embed"]["b"],
                        n_nodes=n_nodes)
    colsums = [cs0]
    for lp in params["layers"]:
        hT = h_blk.reshape(nfb * n_nodes, FB)
        n0 = _sc_agg(hT, srcp, dstr, zeros,
                     n_nodes=n_nodes, n_chunks=n_chunks, m=0)
        n1 = _sc_agg(hT, srcp, dstr, zeros,
                     n_nodes=n_nodes, n_chunks=n_chunks, m=1)
        wr = lp["mlp_lin1"]["W"].reshape(nfb, FB, hid)
        y1a = _k1_partial(h_blk, n0, lp["eps"], wr,
                          n_nodes=n_nodes, m=0, hid=hid)
        y1, st1 = _k1_final(y1a, h_blk, n1, lp["eps"], wr,
                            lp["mlp_lin1"]["b"],
                            n_nodes=n_nodes, m=1, hid=hid)
        y2, st2 = _bn_relu_matmul(y1, st1, lp["mlp_bn1"]["g"], lp["mlp_bn1"]["b"],
                                  lp["mlp_lin2"]["W"], lp["mlp_lin2"]["b"],
                                  n_nodes=n_nodes)
        t_blk, st3 = _bn_relu_snorm(y2, st2, lp["apply_bn"]["g"],
                                    lp["apply_bn"]["b"], snorm_n,
                                    n_nodes=n_nodes)
        h_blk, cs = _bn_relu_residual(t_blk, st3, lp["node_bn"]["g"],
                                      lp["node_bn"]["b"], h_blk,
                                      n_nodes=n_nodes)
        colsums.append(cs)

    pooled_cat = jnp.concatenate(colsums, axis=1)
    w_cat = jnp.concatenate([p["W"] for p in params["pred"]], axis=0)
    b_all = jnp.stack([p["b"] for p in params["pred"]], axis=0)
    return _score(pooled_cat, w_cat, b_all)


# speedup vs baseline: 1.0101x; 1.0101x over previous
import functools

import jax
import jax.numpy as jnp
from jax import lax
from jax.experimental import pallas as pl
from jax.experimental.pallas import tpu as pltpu
from jax.experimental.pallas import tpu_sc as plsc

FB = 128
BM = 5000
CH = 125
N_TILES = 16
N_CORES = 2
BN_EPS = 1e-5


def _sc_agg(hT, srcp, dstr, zeros, *, n_nodes, n_chunks):
    nfb = srcp.shape[0]
    rows_per_tile = (-(-n_nodes // N_TILES) + 7) // 8 * 8
    last_r0 = n_nodes - rows_per_tile
    mesh = plsc.VectorSubcoreMesh(core_axis_name="c", subcore_axis_name="s")

    @functools.partial(
        pl.kernel,
        mesh=mesh,
        out_type=jax.ShapeDtypeStruct((nfb, n_nodes, FB), jnp.float32),
        scratch_types=[
            pltpu.VMEM((n_chunks, CH), jnp.int32),
            pltpu.VMEM((n_chunks, CH), jnp.int32),
            pltpu.VMEM((CH, FB), jnp.float32),
            pltpu.VMEM_SHARED((n_nodes, FB), jnp.float32),
            pltpu.SemaphoreType.DMA,
        ],
    )
    def agg(hT_hbm, srcp_hbm, dst_hbm, z_hbm, out_hbm,
            src_v, dst_v, rows, acc, gsem):
        c = lax.axis_index("c")
        s = lax.axis_index("s")
        r0 = pl.multiple_of(jnp.minimum(s * rows_per_tile, last_r0), 8)
        pltpu.sync_copy(dst_hbm.at[s], dst_v)
        for bi in range(nfb // N_CORES):
            b = c * (nfb // N_CORES) + bi
            pltpu.sync_copy(z_hbm.at[pl.ds(r0, rows_per_tile)],
                            acc.at[pl.ds(r0, rows_per_tile)])
            pltpu.sync_copy(srcp_hbm.at[b, s], src_v)
            plsc.subcore_barrier()

            def body(j, carry):
                pltpu.async_copy(hT_hbm.at[src_v.at[j]], rows, gsem).wait()
                pltpu.sync_copy(rows, acc.at[dst_v.at[j]], add=True)
                return carry

            lax.fori_loop(0, n_chunks, body, 0)
            plsc.subcore_barrier()
            pltpu.sync_copy(acc.at[pl.ds(r0, rows_per_tile)],
                            out_hbm.at[b, pl.ds(r0, rows_per_tile)])
            plsc.subcore_barrier()

    return agg(hT, srcp, dstr, zeros)


def _embed(x, w, b, *, n_nodes):
    nmb = n_nodes // BM
    nfb = w.shape[1] // FB

    def body(x_ref, w_ref, b_ref, o_ref, cs_ref):
        j, i = pl.program_id(0), pl.program_id(1)
        y = jnp.dot(x_ref[...].astype(jnp.bfloat16),
                    w_ref[...].astype(jnp.bfloat16),
                    preferred_element_type=jnp.float32) + b_ref[...]
        y0, y1 = y[:, :FB], y[:, FB:]
        o_ref[...] = jnp.stack([y0, y1])

        @pl.when(i == 0)
        def _():
            cs_ref[...] = jnp.zeros_like(cs_ref)

        cs_ref[...] += jnp.sum(y, axis=0, keepdims=True)

    return pl.pallas_call(
        body,
        grid=(N_CORES, nmb),
        in_specs=[
            pl.BlockSpec((BM, x.shape[1]), lambda j, i: (i, 0)),
            pl.BlockSpec((x.shape[1], 2 * FB), lambda j, i: (0, j)),
            pl.BlockSpec((1, 2 * FB), lambda j, i: (0, j)),
        ],
        out_specs=[
            pl.BlockSpec((2, BM, FB), lambda j, i: (j, i, 0)),
            pl.BlockSpec((1, 2 * FB), lambda j, i: (0, j)),
        ],
        out_shape=[
            jax.ShapeDtypeStruct((nfb, n_nodes, FB), jnp.float32),
            jax.ShapeDtypeStruct((1, nfb * FB), jnp.float32),
        ],
    )(x, w, b.reshape(1, -1))


def _agg_matmul(h_blk, neigh_blk, eps, w, b, *, n_nodes):
    nfb = h_blk.shape[0]
    hid = w.shape[1]
    nmb = n_nodes // BM
    wr = w.reshape(nfb, FB, hid)

    def body(h_ref, n_ref, e_ref, w_ref, b_ref, y_ref, st_ref):
        i, k2 = pl.program_id(0), pl.program_id(1)
        sc = 1.0 + e_ref[0, 0]
        a0 = sc * h_ref[0] + n_ref[0]
        a1 = sc * h_ref[1] + n_ref[1]
        bf = jnp.bfloat16
        part = (jnp.dot(a0.astype(bf), w_ref[0].astype(bf),
                        preferred_element_type=jnp.float32)
                + jnp.dot(a1.astype(bf), w_ref[1].astype(bf),
                          preferred_element_type=jnp.float32))

        @pl.when(k2 == 0)
        def _():
            y_ref[...] = part + b_ref[...]

        @pl.when(k2 > 0)
        def _():
            y_ref[...] += part

        @pl.when(jnp.logical_and(i == 0, k2 == 0))
        def _():
            st_ref[...] = jnp.zeros_like(st_ref)

        @pl.when(k2 == N_CORES - 1)
        def _():
            y = y_ref[...]
            st_ref[0:1, :] += jnp.sum(y, axis=0, keepdims=True)
            st_ref[1:2, :] += jnp.sum(y * y, axis=0, keepdims=True)

    return pl.pallas_call(
        body,
        grid=(nmb, N_CORES),
        in_specs=[
            pl.BlockSpec((2, BM, FB), lambda i, k2: (k2, i, 0)),
            pl.BlockSpec((2, BM, FB), lambda i, k2: (k2, i, 0)),
            pl.BlockSpec((1, 1), lambda i, k2: (0, 0),
                         memory_space=pltpu.SMEM),
            pl.BlockSpec((2, FB, hid), lambda i, k2: (k2, 0, 0)),
            pl.BlockSpec((1, hid), lambda i, k2: (0, 0)),
        ],
        out_specs=[
            pl.BlockSpec((BM, hid), lambda i, k2: (i, 0)),
            pl.BlockSpec((2, hid), lambda i, k2: (0, 0)),
        ],
        out_shape=[
            jax.ShapeDtypeStruct((n_nodes, hid), jnp.float32),
            jax.ShapeDtypeStruct((2, hid), jnp.float32),
        ],
    )(h_blk, neigh_blk, eps.reshape(1, 1), wr, b.reshape(1, -1))


def _bn_relu_matmul(y1, st1, g1, b1, w, b, *, n_nodes):
    hid = w.shape[1]
    nmb = n_nodes // BM

    def body(y1_ref, st_ref, g_ref, bb_ref, w_ref, b_ref, y_ref, st2_ref):
        i = pl.program_id(0)
        n = jnp.float32(n_nodes)
        mean = st_ref[0:1, :] / n
        var = st_ref[1:2, :] / n - mean * mean
        scale = g_ref[...] * lax.rsqrt(var + BN_EPS)
        shift = bb_ref[...] - mean * scale
        z = jnp.maximum(y1_ref[...] * scale + shift, 0.0)
        y = jnp.dot(z.astype(jnp.bfloat16), w_ref[...].astype(jnp.bfloat16),
                    preferred_element_type=jnp.float32) + b_ref[...]
        y_ref[...] = y

        @pl.when(i == 0)
        def _():
            st2_ref[...] = jnp.zeros_like(st2_ref)

        st2_ref[0:1, :] += jnp.sum(y, axis=0, keepdims=True)
        st2_ref[1:2, :] += jnp.sum(y * y, axis=0, keepdims=True)

    return pl.pallas_call(
        body,
        grid=(nmb,),
        in_specs=[
            pl.BlockSpec((BM, hid), lambda i: (i, 0)),
            pl.BlockSpec((2, hid), lambda i: (0, 0)),
            pl.BlockSpec((1, hid), lambda i: (0, 0)),
            pl.BlockSpec((1, hid), lambda i: (0, 0)),
            pl.BlockSpec((hid, hid), lambda i: (0, 0)),
            pl.BlockSpec((1, hid), lambda i: (0, 0)),
        ],
        out_specs=[
            pl.BlockSpec((BM, hid), lambda i: (i, 0)),
            pl.BlockSpec((2, hid), lambda i: (0, 0)),
        ],
        out_shape=[
            jax.ShapeDtypeStruct((n_nodes, hid), jnp.float32),
            jax.ShapeDtypeStruct((2, hid), jnp.float32),
        ],
    )(y1, st1, g1.reshape(1, -1), b1.reshape(1, -1), w, b.reshape(1, -1))


def _bn_relu_snorm(y2, st2, g2, b2, snorm, *, n_nodes):
    hid = y2.shape[1]
    nfb = hid // FB
    nmb = n_nodes // BM

    def body(y_ref, st_ref, g_ref, bb_ref, sn_ref, t_ref, st3_ref):
        j, i = pl.program_id(0), pl.program_id(1)
        n = jnp.float32(n_nodes)
        mean = st_ref[0:1, :] / n
        var = st_ref[1:2, :] / n - mean * mean
        scale = g_ref[...] * lax.rsqrt(var + BN_EPS)
        shift = bb_ref[...] - mean * scale
        t = jnp.maximum(y_ref[...] * scale + shift, 0.0) * sn_ref[...]
        t_ref[...] = t[None]

        @pl.when(i == 0)
        def _():
            st3_ref[...] = jnp.zeros_like(st3_ref)

        st3_ref[0:1, :] += jnp.sum(t, axis=0, keepdims=True)
        st3_ref[1:2, :] += jnp.sum(t * t, axis=0, keepdims=True)

    return pl.pallas_call(
        body,
        grid=(nfb, nmb),
        in_specs=[
            pl.BlockSpec((BM, FB), lambda j, i: (i, j)),
            pl.BlockSpec((2, FB), lambda j, i: (0, j)),
            pl.BlockSpec((1, FB), lambda j, i: (0, j)),
            pl.BlockSpec((1, FB), lambda j, i: (0, j)),
            pl.BlockSpec((BM, 1), lambda j, i: (i, 0)),
        ],
        out_specs=[
            pl.BlockSpec((1, BM, FB), lambda j, i: (j, i, 0)),
            pl.BlockSpec((2, FB), lambda j, i: (0, j)),
        ],
        out_shape=[
            jax.ShapeDtypeStruct((nfb, n_nodes, FB), jnp.float32),
            jax.ShapeDtypeStruct((2, hid), jnp.float32),
        ],
    )(y2, st2, g2.reshape(1, -1), b2.reshape(1, -1), snorm)


def _bn_relu_residual(t_blk, st3, g3, b3, h_blk, *, n_nodes):
    nfb = t_blk.shape[0]
    nmb = n_nodes // BM

    def body(t_ref, st_ref, g_ref, bb_ref, h_ref, o_ref, cs_ref):
        j, i = pl.program_id(0), pl.program_id(1)
        n = jnp.float32(n_nodes)
        mean = st_ref[0:1, :] / n
        var = st_ref[1:2, :] / n - mean * mean
        scale = g_ref[...] * lax.rsqrt(var + BN_EPS)
        shift = bb_ref[...] - mean * scale
        h0 = h_ref[0] + jnp.maximum(
            t_ref[0] * scale[:, :FB] + shift[:, :FB], 0.0)
        h1 = h_ref[1] + jnp.maximum(
            t_ref[1] * scale[:, FB:] + shift[:, FB:], 0.0)
        o_ref[...] = jnp.stack([h0, h1])

        @pl.when(i == 0)
        def _():
            cs_ref[...] = jnp.zeros_like(cs_ref)

        cs_ref[...] += jnp.concatenate(
            [jnp.sum(h0, axis=0, keepdims=True),
             jnp.sum(h1, axis=0, keepdims=True)], axis=1)

    return pl.pallas_call(
        body,
        grid=(N_CORES, nmb),
        in_specs=[
            pl.BlockSpec((2, BM, FB), lambda j, i: (j, i, 0)),
            pl.BlockSpec((2, 2 * FB), lambda j, i: (0, j)),
            pl.BlockSpec((1, 2 * FB), lambda j, i: (0, j)),
            pl.BlockSpec((1, 2 * FB), lambda j, i: (0, j)),
            pl.BlockSpec((2, BM, FB), lambda j, i: (j, i, 0)),
        ],
        out_specs=[
            pl.BlockSpec((2, BM, FB), lambda j, i: (j, i, 0)),
            pl.BlockSpec((1, 2 * FB), lambda j, i: (0, j)),
        ],
        out_shape=[
            jax.ShapeDtypeStruct((nfb, n_nodes, FB), jnp.float32),
            jax.ShapeDtypeStruct((1, nfb * FB), jnp.float32),
        ],
    )(t_blk, st3, g3.reshape(1, -1), b3.reshape(1, -1), h_blk)


def _score(pooled_cat, w_cat, b_all):
    ncls = w_cat.shape[1]

    def body(p_ref, w_ref, b_ref, o_ref):
        o_ref[...] = (
            jnp.dot(p_ref[...], w_ref[...], preferred_element_type=jnp.float32)
            + jnp.sum(b_ref[...], axis=0, keepdims=True))

    return pl.pallas_call(
        body,
        out_shape=jax.ShapeDtypeStruct((1, ncls), jnp.float32),
    )(pooled_cat, w_cat, b_all)


def kernel(h, edge_index, e, snorm_n, snorm_e, params):
    n_nodes = h.shape[0]
    n_edges = edge_index.shape[1]
    hid = params["embed"]["W"].shape[1]
    nfb = hid // FB
    ept = n_edges // N_TILES
    n_chunks = ept // CH

    src = edge_index[0]
    dst = edge_index[1]
    offs = (jnp.arange(nfb, dtype=jnp.int32) * n_nodes)[:, None]
    srcp = (src[None, :] + offs).reshape(nfb, N_TILES, n_chunks, CH)
    dstr = dst.reshape(N_TILES, n_chunks, CH)
    zeros = jnp.zeros((n_nodes, FB), jnp.float32)

    h_blk, cs0 = _embed(h, params["embed"]["W"], params["embed"]["b"],
                        n_nodes=n_nodes)
    colsums = [cs0]
    for lp in params["layers"]:
        hT = h_blk.reshape(nfb * n_nodes, FB)
        neigh_blk = _sc_agg(hT, srcp, dstr, zeros,
                            n_nodes=n_nodes, n_chunks=n_chunks)
        y1, st1 = _agg_matmul(h_blk, neigh_blk, lp["eps"],
                              lp["mlp_lin1"]["W"], lp["mlp_lin1"]["b"],
                              n_nodes=n_nodes)
        y2, st2 = _bn_relu_matmul(y1, st1, lp["mlp_bn1"]["g"], lp["mlp_bn1"]["b"],
                                  lp["mlp_lin2"]["W"], lp["mlp_lin2"]["b"],
                                  n_nodes=n_nodes)
        t_blk, st3 = _bn_relu_snorm(y2, st2, lp["apply_bn"]["g"],
                                    lp["apply_bn"]["b"], snorm_n,
                                    n_nodes=n_nodes)
        h_blk, cs = _bn_relu_residual(t_blk, st3, lp["node_bn"]["g"],
                                      lp["node_bn"]["b"], h_blk,
                                      n_nodes=n_nodes)
        colsums.append(cs)

    pooled_cat = jnp.concatenate(colsums, axis=1)
    w_cat = jnp.concatenate([p["W"] for p in params["pred"]], axis=0)
    b_all = jnp.stack([p["b"] for p in params["pred"]], axis=0)
    return _score(pooled_cat, w_cat, b_all)


# speedup vs baseline: 1.1639x; 1.1522x over previous
import functools

import jax
import jax.numpy as jnp
from jax import lax
from jax.experimental import pallas as pl
from jax.experimental.pallas import tpu as pltpu
from jax.experimental.pallas import tpu_sc as plsc

FB = 128
BM = 5000
CH = 125
N_TILES = 16
N_CORES = 2
BN_EPS = 1e-5


def _sc_agg(hT, srcp, dstr, zeros, *, n_nodes, n_half):
    nfb = srcp.shape[0]
    rows_per_tile = (-(-n_nodes // N_TILES) + 7) // 8 * 8
    last_r0 = n_nodes - rows_per_tile
    mesh = plsc.VectorSubcoreMesh(core_axis_name="c", subcore_axis_name="s")

    @functools.partial(
        pl.kernel,
        mesh=mesh,
        out_type=jax.ShapeDtypeStruct((nfb, n_nodes, FB), jnp.float32),
        scratch_types=[
            pltpu.VMEM((n_half, CH), jnp.int32),
            pltpu.VMEM((n_half, CH), jnp.int32),
            [pltpu.VMEM((CH, FB), jnp.float32)] * 2,
            pltpu.VMEM_SHARED((n_nodes, FB), jnp.float32),
            [pltpu.SemaphoreType.DMA] * 2,
            [pltpu.SemaphoreType.DMA] * 2,
        ],
    )
    def agg(hT_hbm, srcp_hbm, dst_hbm, z_hbm, out_hbm,
            src_v, dst_v, rows, acc, gsems, ssems):
        c = lax.axis_index("c")
        s = lax.axis_index("s")
        r0 = pl.multiple_of(jnp.minimum(s * rows_per_tile, last_r0), 8)
        for bi in range(nfb // N_CORES):
            b = c * (nfb // N_CORES) + bi
            pltpu.sync_copy(z_hbm.at[pl.ds(r0, rows_per_tile)],
                            acc.at[pl.ds(r0, rows_per_tile)])
            plsc.subcore_barrier()
            for half in range(2):
                pltpu.sync_copy(srcp_hbm.at[b, s, half], src_v)
                pltpu.sync_copy(dst_hbm.at[s, half], dst_v)
                for k in range(2):
                    pltpu.async_copy(hT_hbm.at[src_v.at[k]], rows[k],
                                     gsems[k])

                def body(p, carry):
                    for k in range(2):
                        j = 2 * p + k
                        pltpu.make_async_copy(hT_hbm.at[src_v.at[j]],
                                              rows[k], gsems[k]).wait()
                        pltpu.async_copy(rows[k], acc.at[dst_v.at[j]],
                                         ssems[k], add=True)
                    for k in range(2):
                        j = 2 * p + k
                        pltpu.make_async_copy(rows[k], acc.at[dst_v.at[j]],
                                              ssems[k]).wait()

                        @pl.when(j + 2 < n_half)
                        def _():
                            pltpu.async_copy(hT_hbm.at[src_v.at[j + 2]],
                                             rows[k], gsems[k])
                    return carry

                lax.fori_loop(0, n_half // 2, body, 0)
            plsc.subcore_barrier()
            pltpu.sync_copy(acc.at[pl.ds(r0, rows_per_tile)],
                            out_hbm.at[b, pl.ds(r0, rows_per_tile)])
            plsc.subcore_barrier()

    return agg(hT, srcp, dstr, zeros)


def _embed(x, w, b, *, n_nodes):
    nmb = n_nodes // BM
    nfb = w.shape[1] // FB

    def body(x_ref, w_ref, b_ref, o_ref, cs_ref):
        j, i = pl.program_id(0), pl.program_id(1)
        y = jnp.dot(x_ref[...].astype(jnp.bfloat16),
                    w_ref[...].astype(jnp.bfloat16),
                    preferred_element_type=jnp.float32) + b_ref[...]
        y0, y1 = y[:, :FB], y[:, FB:]
        o_ref[...] = jnp.stack([y0, y1])

        @pl.when(i == 0)
        def _():
            cs_ref[...] = jnp.zeros_like(cs_ref)

        cs_ref[...] += jnp.sum(y, axis=0, keepdims=True)

    return pl.pallas_call(
        body,
        grid=(N_CORES, nmb),
        in_specs=[
            pl.BlockSpec((BM, x.shape[1]), lambda j, i: (i, 0)),
            pl.BlockSpec((x.shape[1], 2 * FB), lambda j, i: (0, j)),
            pl.BlockSpec((1, 2 * FB), lambda j, i: (0, j)),
        ],
        out_specs=[
            pl.BlockSpec((2, BM, FB), lambda j, i: (j, i, 0)),
            pl.BlockSpec((1, 2 * FB), lambda j, i: (0, j)),
        ],
        out_shape=[
            jax.ShapeDtypeStruct((nfb, n_nodes, FB), jnp.float32),
            jax.ShapeDtypeStruct((1, nfb * FB), jnp.float32),
        ],
    )(x, w, b.reshape(1, -1))


def _agg_matmul(h_blk, neigh_blk, eps, w, b, *, n_nodes):
    nfb = h_blk.shape[0]
    hid = w.shape[1]
    nmb = n_nodes // BM
    wr = w.reshape(nfb, FB, hid)

    def body(h_ref, n_ref, e_ref, w_ref, b_ref, y_ref, st_ref):
        i, k2 = pl.program_id(0), pl.program_id(1)
        sc = 1.0 + e_ref[0, 0]
        a0 = sc * h_ref[0] + n_ref[0]
        a1 = sc * h_ref[1] + n_ref[1]
        bf = jnp.bfloat16
        part = (jnp.dot(a0.astype(bf), w_ref[0].astype(bf),
                        preferred_element_type=jnp.float32)
                + jnp.dot(a1.astype(bf), w_ref[1].astype(bf),
                          preferred_element_type=jnp.float32))

        @pl.when(k2 == 0)
        def _():
            y_ref[...] = part + b_ref[...]

        @pl.when(k2 > 0)
        def _():
            y_ref[...] += part

        @pl.when(jnp.logical_and(i == 0, k2 == 0))
        def _():
            st_ref[...] = jnp.zeros_like(st_ref)

        @pl.when(k2 == N_CORES - 1)
        def _():
            y = y_ref[...]
            st_ref[0:1, :] += jnp.sum(y, axis=0, keepdims=True)
            st_ref[1:2, :] += jnp.sum(y * y, axis=0, keepdims=True)

    return pl.pallas_call(
        body,
        grid=(nmb, N_CORES),
        in_specs=[
            pl.BlockSpec((2, BM, FB), lambda i, k2: (k2, i, 0)),
            pl.BlockSpec((2, BM, FB), lambda i, k2: (k2, i, 0)),
            pl.BlockSpec((1, 1), lambda i, k2: (0, 0),
                         memory_space=pltpu.SMEM),
            pl.BlockSpec((2, FB, hid), lambda i, k2: (k2, 0, 0)),
            pl.BlockSpec((1, hid), lambda i, k2: (0, 0)),
        ],
        out_specs=[
            pl.BlockSpec((BM, hid), lambda i, k2: (i, 0)),
            pl.BlockSpec((2, hid), lambda i, k2: (0, 0)),
        ],
        out_shape=[
            jax.ShapeDtypeStruct((n_nodes, hid), jnp.float32),
            jax.ShapeDtypeStruct((2, hid), jnp.float32),
        ],
    )(h_blk, neigh_blk, eps.reshape(1, 1), wr, b.reshape(1, -1))


def _bn_relu_matmul(y1, st1, g1, b1, w, b, *, n_nodes):
    hid = w.shape[1]
    nmb = n_nodes // BM

    def body(y1_ref, st_ref, g_ref, bb_ref, w_ref, b_ref, y_ref, st2_ref):
        i = pl.program_id(0)
        n = jnp.float32(n_nodes)
        mean = st_ref[0:1, :] / n
        var = st_ref[1:2, :] / n - mean * mean
        scale = g_ref[...] * lax.rsqrt(var + BN_EPS)
        shift = bb_ref[...] - mean * scale
        z = jnp.maximum(y1_ref[...] * scale + shift, 0.0)
        y = jnp.dot(z.astype(jnp.bfloat16), w_ref[...].astype(jnp.bfloat16),
                    preferred_element_type=jnp.float32) + b_ref[...]
        y_ref[...] = y

        @pl.when(i == 0)
        def _():
            st2_ref[...] = jnp.zeros_like(st2_ref)

        st2_ref[0:1, :] += jnp.sum(y, axis=0, keepdims=True)
        st2_ref[1:2, :] += jnp.sum(y * y, axis=0, keepdims=True)

    return pl.pallas_call(
        body,
        grid=(nmb,),
        in_specs=[
            pl.BlockSpec((BM, hid), lambda i: (i, 0)),
            pl.BlockSpec((2, hid), lambda i: (0, 0)),
            pl.BlockSpec((1, hid), lambda i: (0, 0)),
            pl.BlockSpec((1, hid), lambda i: (0, 0)),
            pl.BlockSpec((hid, hid), lambda i: (0, 0)),
            pl.BlockSpec((1, hid), lambda i: (0, 0)),
        ],
        out_specs=[
            pl.BlockSpec((BM, hid), lambda i: (i, 0)),
            pl.BlockSpec((2, hid), lambda i: (0, 0)),
        ],
        out_shape=[
            jax.ShapeDtypeStruct((n_nodes, hid), jnp.float32),
            jax.ShapeDtypeStruct((2, hid), jnp.float32),
        ],
    )(y1, st1, g1.reshape(1, -1), b1.reshape(1, -1), w, b.reshape(1, -1))


def _bn_relu_snorm(y2, st2, g2, b2, snorm, *, n_nodes):
    hid = y2.shape[1]
    nfb = hid // FB
    nmb = n_nodes // BM

    def body(y_ref, st_ref, g_ref, bb_ref, sn_ref, t_ref, st3_ref):
        j, i = pl.program_id(0), pl.program_id(1)
        n = jnp.float32(n_nodes)
        mean = st_ref[0:1, :] / n
        var = st_ref[1:2, :] / n - mean * mean
        scale = g_ref[...] * lax.rsqrt(var + BN_EPS)
        shift = bb_ref[...] - mean * scale
        t = jnp.maximum(y_ref[...] * scale + shift, 0.0) * sn_ref[...]
        t_ref[...] = t[None]

        @pl.when(i == 0)
        def _():
            st3_ref[...] = jnp.zeros_like(st3_ref)

        st3_ref[0:1, :] += jnp.sum(t, axis=0, keepdims=True)
        st3_ref[1:2, :] += jnp.sum(t * t, axis=0, keepdims=True)

    return pl.pallas_call(
        body,
        grid=(nfb, nmb),
        in_specs=[
            pl.BlockSpec((BM, FB), lambda j, i: (i, j)),
            pl.BlockSpec((2, FB), lambda j, i: (0, j)),
            pl.BlockSpec((1, FB), lambda j, i: (0, j)),
            pl.BlockSpec((1, FB), lambda j, i: (0, j)),
            pl.BlockSpec((BM, 1), lambda j, i: (i, 0)),
        ],
        out_specs=[
            pl.BlockSpec((1, BM, FB), lambda j, i: (j, i, 0)),
            pl.BlockSpec((2, FB), lambda j, i: (0, j)),
        ],
        out_shape=[
            jax.ShapeDtypeStruct((nfb, n_nodes, FB), jnp.float32),
            jax.ShapeDtypeStruct((2, hid), jnp.float32),
        ],
    )(y2, st2, g2.reshape(1, -1), b2.reshape(1, -1), snorm)


def _bn_relu_residual(t_blk, st3, g3, b3, h_blk, *, n_nodes):
    nfb = t_blk.shape[0]
    nmb = n_nodes // BM

    def body(t_ref, st_ref, g_ref, bb_ref, h_ref, o_ref, cs_ref):
        j, i = pl.program_id(0), pl.program_id(1)
        n = jnp.float32(n_nodes)
        mean = st_ref[0:1, :] / n
        var = st_ref[1:2, :] / n - mean * mean
        scale = g_ref[...] * lax.rsqrt(var + BN_EPS)
        shift = bb_ref[...] - mean * scale
        h0 = h_ref[0] + jnp.maximum(
            t_ref[0] * scale[:, :FB] + shift[:, :FB], 0.0)
        h1 = h_ref[1] + jnp.maximum(
            t_ref[1] * scale[:, FB:] + shift[:, FB:], 0.0)
        o_ref[...] = jnp.stack([h0, h1])

        @pl.when(i == 0)
        def _():
            cs_ref[...] = jnp.zeros_like(cs_ref)

        cs_ref[...] += jnp.concatenate(
            [jnp.sum(h0, axis=0, keepdims=True),
             jnp.sum(h1, axis=0, keepdims=True)], axis=1)

    return pl.pallas_call(
        body,
        grid=(N_CORES, nmb),
        in_specs=[
            pl.BlockSpec((2, BM, FB), lambda j, i: (j, i, 0)),
            pl.BlockSpec((2, 2 * FB), lambda j, i: (0, j)),
            pl.BlockSpec((1, 2 * FB), lambda j, i: (0, j)),
            pl.BlockSpec((1, 2 * FB), lambda j, i: (0, j)),
            pl.BlockSpec((2, BM, FB), lambda j, i: (j, i, 0)),
        ],
        out_specs=[
            pl.BlockSpec((2, BM, FB), lambda j, i: (j, i, 0)),
            pl.BlockSpec((1, 2 * FB), lambda j, i: (0, j)),
        ],
        out_shape=[
            jax.ShapeDtypeStruct((nfb, n_nodes, FB), jnp.float32),
            jax.ShapeDtypeStruct((1, nfb * FB), jnp.float32),
        ],
    )(t_blk, st3, g3.reshape(1, -1), b3.reshape(1, -1), h_blk)


def _score(pooled_cat, w_cat, b_all):
    ncls = w_cat.shape[1]

    def body(p_ref, w_ref, b_ref, o_ref):
        o_ref[...] = (
            jnp.dot(p_ref[...], w_ref[...], preferred_element_type=jnp.float32)
            + jnp.sum(b_ref[...], axis=0, keepdims=True))

    return pl.pallas_call(
        body,
        out_shape=jax.ShapeDtypeStruct((1, ncls), jnp.float32),
    )(pooled_cat, w_cat, b_all)


def kernel(h, edge_index, e, snorm_n, snorm_e, params):
    n_nodes = h.shape[0]
    n_edges = edge_index.shape[1]
    hid = params["embed"]["W"].shape[1]
    nfb = hid // FB
    ept = n_edges // N_TILES
    n_half = ept // (2 * CH)

    src = edge_index[0]
    dst = edge_index[1]
    offs = (jnp.arange(nfb, dtype=jnp.int32) * n_nodes)[:, None]
    srcp = (src[None, :] + offs).reshape(nfb, N_TILES, 2, n_half, CH)
    dstr = dst.reshape(N_TILES, 2, n_half, CH)
    zeros = jnp.zeros((n_nodes, FB), jnp.float32)

    h_blk, cs0 = _embed(h, params["embed"]["W"], params["embed"]["b"],
                        n_nodes=n_nodes)
    colsums = [cs0]
    for lp in params["layers"]:
        hT = h_blk.reshape(nfb * n_nodes, FB)
        neigh_blk = _sc_agg(hT, srcp, dstr, zeros,
                            n_nodes=n_nodes, n_half=n_half)
        y1, st1 = _agg_matmul(h_blk, neigh_blk, lp["eps"],
                              lp["mlp_lin1"]["W"], lp["mlp_lin1"]["b"],
                              n_nodes=n_nodes)
        y2, st2 = _bn_relu_matmul(y1, st1, lp["mlp_bn1"]["g"], lp["mlp_bn1"]["b"],
                                  lp["mlp_lin2"]["W"], lp["mlp_lin2"]["b"],
                                  n_nodes=n_nodes)
        t_blk, st3 = _bn_relu_snorm(y2, st2, lp["apply_bn"]["g"],
                                    lp["apply_bn"]["b"], snorm_n,
                                    n_nodes=n_nodes)
        h_blk, cs = _bn_relu_residual(t_blk, st3, lp["node_bn"]["g"],
                                      lp["node_bn"]["b"], h_blk,
                                      n_nodes=n_nodes)
        colsums.append(cs)

    pooled_cat = jnp.concatenate(colsums, axis=1)
    w_cat = jnp.concatenate([p["W"] for p in params["pred"]], axis=0)
    b_all = jnp.stack([p["b"] for p in params["pred"]], axis=0)
    return _score(pooled_cat, w_cat, b_all)
